# baseline (speedup 1.0000x reference)
import functools

import jax
import jax.numpy as jnp
from jax import lax
from jax.experimental import pallas as pl
from jax.experimental.pallas import tpu as pltpu
from jax.experimental.pallas import tpu_sc as plsc

_NUM_CLUSTER = 1024
_NUM_HEAD = 16
_KEY_LEN = 3 * _NUM_CLUSTER
_ROW_WORDS = _KEY_LEN * _NUM_HEAD
_TABLE_WORDS = (4 * _NUM_CLUSTER - 1) * _NUM_HEAD

_NUM_WORKERS = 32
_ROWS_PER_WORKER = _NUM_CLUSTER // _NUM_WORKERS


def _sc_broadcast(bias_params):
    mesh = plsc.VectorSubcoreMesh(core_axis_name="c", subcore_axis_name="s")

    @functools.partial(
        pl.kernel,
        mesh=mesh,
        out_type=jax.ShapeDtypeStruct(
            (_NUM_CLUSTER, _KEY_LEN, _NUM_HEAD), jnp.float32
        ),
        scratch_types=[
            pltpu.VMEM((4 * _NUM_CLUSTER - 1, _NUM_HEAD), jnp.float32),
            pltpu.SemaphoreType.DMA,
        ],
        compiler_params=pltpu.CompilerParams(use_tc_tiling_on_sc=False),
    )
    def k(table_hbm, out_hbm, table_v, sem):
        wid = lax.axis_index("s") * 2 + lax.axis_index("c")
        pltpu.sync_copy(table_hbm, table_v)
        base = wid * _ROWS_PER_WORKER
        copies = []
        for i in range(_ROWS_PER_WORKER):
            q = base + i
            row_lo = (_NUM_CLUSTER - 1) - q
            copy = pltpu.make_async_copy(
                table_v.at[pl.ds(row_lo, _KEY_LEN), :],
                out_hbm.at[q],
                sem,
            )
            copy.start()
            copies.append(copy)
        for copy in copies:
            copy.wait()

    return k(bias_params)


def kernel(bias_params):
    return _sc_broadcast(bias_params)


# speedup vs baseline: 14.0464x; 14.0464x over previous
import functools

import jax
import jax.numpy as jnp
from jax import lax
from jax.experimental import pallas as pl
from jax.experimental.pallas import tpu as pltpu
from jax.experimental.pallas import tpu_sc as plsc

_Q = 1024
_H = 16
_K = 3 * _Q
_P = 4 * _Q - 1
_HB, _HI = 2, 8
_KB, _KI = 24, 128
_PC = 4096

_NUM_WORKERS = 32
_ROWS_PER_WORKER = _Q // _NUM_WORKERS


def _sc_broadcast(shifted_tables):
    mesh = plsc.VectorSubcoreMesh(core_axis_name="c", subcore_axis_name="s")

    @functools.partial(
        pl.kernel,
        mesh=mesh,
        out_type=jax.ShapeDtypeStruct((_Q, _HB, _KB, _HI, _KI), jnp.float32),
        scratch_types=[
            pltpu.VMEM((_H, _PC), jnp.float32),
            pltpu.SemaphoreType.DMA,
        ],
        compiler_params=pltpu.CompilerParams(use_tc_tiling_on_sc=False),
    )
    def k(shift_hbm, out_hbm, slab_v, sem):
        wid = lax.axis_index("s") * 2 + lax.axis_index("c")
        r = ((_Q - 1) - wid) % 8
        pltpu.sync_copy(shift_hbm.at[r], slab_v)

        def per_q(t, carry):
            q = wid + _NUM_WORKERS * t
            w = (_Q - 1) - q
            wa = pl.multiple_of(w - r, 8)
            copies = []
            for hb in range(_HB):
                for kb in range(_KB):
                    copy = pltpu.make_async_copy(
                        slab_v.at[pl.ds(hb * _HI, _HI), pl.ds(wa + kb * _KI, _KI)],
                        out_hbm.at[q, hb, kb],
                        sem,
                    )
                    copy.start()
                    copies.append(copy)
            for copy in copies:
                copy.wait()
            return carry

        lax.fori_loop(0, _ROWS_PER_WORKER, per_q, 0)

    return k(shifted_tables)


def kernel(bias_params):
    tt = jnp.pad(bias_params.T, ((0, 0), (0, _PC + 7 - _P)))
    shifted = jnp.stack([tt[:, s : s + _PC] for s in range(8)])
    out5 = _sc_broadcast(shifted)
    return out5.transpose(0, 2, 4, 1, 3).reshape(_Q, _K, _H)
